# Initial kernel scaffold; baseline (speedup 1.0000x reference)
import jax
import jax.numpy as jnp
from jax.experimental import pallas as pl


def kernel(quant_fn, x, emb_idx, W1, b1, W2, b2):
    raise NotImplementedError("write your pallas kernel here")


# speedup vs baseline: 14.6635x; 14.6635x over previous
import jax
import jax.numpy as jnp
from jax.experimental import pallas as pl


def _moe_body(eidx_ref, x_ref, W1_ref, b1_ref, W2_ref, b2_ref, out_ref):
    rows = x_ref[...]
    ei = eidx_ref[...]
    E = W1_ref.shape[0]
    acc = jnp.zeros(out_ref.shape, jnp.float32)
    dn = (((1,), (1,)), ((), ()))
    for e in range(E):
        h = jax.lax.dot_general(rows, W1_ref[e], dn,
                                preferred_element_type=jnp.float32)
        h = jax.nn.relu(h + b1_ref[e:e + 1, :])
        y = jax.lax.dot_general(h, W2_ref[e], dn,
                                preferred_element_type=jnp.float32)
        y = y + b2_ref[e:e + 1, :]
        acc = jnp.where(ei == e, y, acc)
    out_ref[...] = acc


def kernel(quant_fn, x, emb_idx, W1, b1, W2, b2):
    del quant_fn
    B, X = x.shape
    E, H, _ = W1.shape
    O = W2.shape[1]
    TB = 256
    eidx2 = emb_idx.reshape(B, 1)
    return pl.pallas_call(
        _moe_body,
        grid=(B // TB,),
        in_specs=[
            pl.BlockSpec((TB, 1), lambda i: (i, 0)),
            pl.BlockSpec((TB, X), lambda i: (i, 0)),
            pl.BlockSpec((E, H, X), lambda i: (0, 0, 0)),
            pl.BlockSpec((E, H), lambda i: (0, 0)),
            pl.BlockSpec((E, O, H), lambda i: (0, 0, 0)),
            pl.BlockSpec((E, O), lambda i: (0, 0)),
        ],
        out_specs=pl.BlockSpec((TB, O), lambda i: (i, 0)),
        out_shape=jax.ShapeDtypeStruct((B, O), jnp.float32),
    )(eidx2, x, W1, b1, W2, b2)
